# baseline (speedup 1.0000x reference)
import functools

import jax
import jax.numpy as jnp
from jax.experimental import pallas as pl
from jax.experimental.pallas import tpu as pltpu

_CHUNK = 2048


def _chamfer_body(q_ref, r_ref, out_ref, *, q_total):
    q = q_ref[:, :]
    d = q.shape[1]
    tq = q.shape[0]
    q2 = jnp.sum(q * q, axis=1)
    qm2 = (q * -2.0).astype(jnp.bfloat16)

    r_total = r_ref.shape[0]
    ones8 = jnp.ones((8, d), jnp.bfloat16)
    row_min = None
    for c in range(r_total // _CHUNK):
        r_c = r_ref[c * _CHUNK:(c + 1) * _CHUNK, :]
        rb_c = r_c.astype(jnp.bfloat16)
        r2_c = jax.lax.dot_general(
            ones8, (r_c * r_c).astype(jnp.bfloat16),
            dimension_numbers=(((1,), (1,)), ((), ())),
            preferred_element_type=jnp.float32,
        )
        t_c = jax.lax.dot_general(
            qm2, rb_c,
            dimension_numbers=(((1,), (1,)), ((), ())),
            preferred_element_type=jnp.float32,
        )
        t3 = t_c.reshape(tq // 8, 8, _CHUNK)
        m_c = jnp.min(t3 + r2_c[None, :, :], axis=2)
        row_min = m_c if row_min is None else jnp.minimum(row_min, m_c)

    total = jnp.sum(row_min) + jnp.sum(q2)
    out_ref[:, :] = total.reshape(1, 1) / q_total


def kernel(query, ref):
    q_total, d = query.shape
    r_total, _ = ref.shape

    body = functools.partial(_chamfer_body, q_total=float(q_total))
    out = pl.pallas_call(
        body,
        in_specs=[
            pl.BlockSpec((q_total, d), lambda: (0, 0)),
            pl.BlockSpec((r_total, d), lambda: (0, 0)),
        ],
        out_specs=pl.BlockSpec((1, 1), lambda: (0, 0)),
        out_shape=jax.ShapeDtypeStruct((1, 1), jnp.float32),
        compiler_params=pltpu.CompilerParams(
            vmem_limit_bytes=128 * 1024 * 1024),
    )(query, ref)
    return out[0, 0]


# speedup vs baseline: 1.1575x; 1.1575x over previous
import functools

import jax
import jax.numpy as jnp
from jax.experimental import pallas as pl
from jax.experimental.pallas import tpu as pltpu


def _chamfer_body(q_ref, r_ref, out_ref, raug_scratch, *, n_q_tiles, q_total):
    i = pl.program_id(0)

    @pl.when(i == 0)
    def _prep():
        r = r_ref[:, :]
        d = r.shape[1]
        raug_scratch[:, :d] = r.astype(jnp.bfloat16)
        raug_scratch[:, d:] = (r * r).astype(jnp.bfloat16)

    q = q_ref[:, :]
    q2 = jnp.sum(q * q, axis=1)
    q_aug = jnp.concatenate(
        [q * -2.0, jnp.ones_like(q)], axis=1).astype(jnp.bfloat16)

    t = jax.lax.dot_general(
        q_aug,
        raug_scratch[:, :],
        dimension_numbers=(((1,), (1,)), ((), ())),
        preferred_element_type=jnp.float32,
    )

    row_min = jnp.min(t, axis=1) + q2
    tile_sum = jnp.sum(row_min).reshape(1, 1)

    @pl.when(i == 0)
    def _init():
        out_ref[:, :] = tile_sum

    @pl.when(i > 0)
    def _acc():
        out_ref[:, :] = out_ref[:, :] + tile_sum

    @pl.when(i == n_q_tiles - 1)
    def _finish():
        out_ref[:, :] = out_ref[:, :] / q_total


def kernel(query, ref):
    q_total, d = query.shape
    r_total, _ = ref.shape

    tile_q = 1024 if q_total % 1024 == 0 else q_total
    n_q_tiles = q_total // tile_q

    body = functools.partial(_chamfer_body, n_q_tiles=n_q_tiles,
                             q_total=float(q_total))
    out = pl.pallas_call(
        body,
        grid=(n_q_tiles,),
        in_specs=[
            pl.BlockSpec((tile_q, d), lambda i: (i, 0)),
            pl.BlockSpec((r_total, d), lambda i: (0, 0)),
        ],
        out_specs=pl.BlockSpec((1, 1), lambda i: (0, 0)),
        out_shape=jax.ShapeDtypeStruct((1, 1), jnp.float32),
        scratch_shapes=[
            pltpu.VMEM((r_total, 2 * d), jnp.bfloat16),
        ],
        compiler_params=pltpu.CompilerParams(
            vmem_limit_bytes=128 * 1024 * 1024),
    )(query, ref)
    return out[0, 0]
